# baseline (speedup 1.0000x reference)
import functools

import jax
import jax.numpy as jnp
from jax import lax
from jax.experimental import pallas as pl
from jax.experimental.pallas import tpu as pltpu
from jax.experimental.pallas import tpu_sc as plsc

N_NODES = 10000
D = 128
N_EDGES = 320000

NC = 2
NS = 16

BLOCK = 4096
EPT = 20480
NBLK = EPT // BLOCK
E_PAD = NS * EPT

SPAN = 5056
AGG_ROWS = 5120
DUMMY = SPAN
OUT_ROWS = NC * SPAN

CHUNK = 128
LCAP = 20768
TRASH = 20752
MAXC = 160

_mesh = plsc.VectorSubcoreMesh(core_axis_name="c", subcore_axis_name="s")


@functools.partial(
    pl.kernel,
    out_type=jax.ShapeDtypeStruct((OUT_ROWS, D), jnp.float32),
    mesh=_mesh,
    compiler_params=pltpu.CompilerParams(needs_layout_passes=False),
    scratch_types=[
        pltpu.VMEM((BLOCK,), jnp.int32),
        pltpu.VMEM((BLOCK,), jnp.int32),
        pltpu.VMEM((LCAP,), jnp.int32),
        pltpu.VMEM((LCAP,), jnp.int32),
        pltpu.VMEM((2, CHUNK), jnp.int32),
        pltpu.VMEM((2, CHUNK, D), jnp.float32),
        pltpu.VMEM_SHARED((AGG_ROWS, D), jnp.float32),
        pltpu.SemaphoreType.DMA,
        pltpu.SemaphoreType.DMA,
        pltpu.SemaphoreType.DMA,
        pltpu.SemaphoreType.DMA,
    ],
)
def _sc_aggregate(x_hbm, src_hbm, dst_hbm, out_hbm,
                  sblk, dblk, slist, dlist, dstb, rows_v, agg_sh,
                  gsem0, gsem1, ssem0, ssem1):
    cid = lax.axis_index("c")
    sid = lax.axis_index("s")
    gsem = (gsem0, gsem1)
    ssem = (ssem0, ssem1)
    lo = cid * SPAN

    zero16 = jnp.zeros((16,), jnp.float32)

    def _zero_row(r, carry):
        for j in range(D // 16):
            rows_v[0, r, pl.ds(j * 16, 16)] = zero16
        return carry

    lax.fori_loop(0, CHUNK, _zero_row, 0)

    zbase = sid * (AGG_ROWS // NS)
    zfull, zrem = divmod(AGG_ROWS // NS, CHUNK)
    for k in range(zfull):
        pltpu.sync_copy(rows_v.at[0],
                        agg_sh.at[pl.ds(zbase + k * CHUNK, CHUNK)])
    if zrem:
        pltpu.sync_copy(rows_v.at[0, pl.ds(0, zrem)],
                        agg_sh.at[pl.ds(zbase + zfull * CHUNK, zrem)])

    sbase = sid * EPT

    lane = lax.iota(jnp.int32, 16)

    def _store_pair(pos, s, dl):
        plsc.store_scatter(slist, [pos], s)
        plsc.store_scatter(dlist, [pos], dl)

    ones16 = jnp.ones((16,), jnp.int32)
    zeros16 = jnp.zeros((16,), jnp.int32)

    def _scan_group(g, cntv):
        d = dblk[pl.ds(g * 16, 16)]
        s = sblk[pl.ds(g * 16, 16)]
        dl = d - lo
        m = (dl >= 0) & (dl < SPAN)
        mi = jnp.where(m, ones16, zeros16)
        pos = jnp.where(m, plsc.cumsum(mi) - mi + cntv, TRASH + lane)
        _store_pair(pos, s, dl)
        return cntv + plsc.all_reduce_population_count(m)

    cntv = zeros16
    for blk in range(NBLK):
        off = sbase + blk * BLOCK
        pltpu.sync_copy(src_hbm.at[pl.ds(off, BLOCK)], sblk)
        pltpu.sync_copy(dst_hbm.at[pl.ds(off, BLOCK)], dblk)
        cntv = lax.fori_loop(0, BLOCK // 16, _scan_group, cntv)

    padv_d = jnp.full((16,), DUMMY, jnp.int32)
    for k in range(2 * (CHUNK // 16) + 1):
        _store_pair(cntv + 16 * k + lane, zeros16, padv_d)
    cnt = cntv[0]
    nchunks = jnp.maximum(2 * ((cnt + 2 * CHUNK - 1) // (2 * CHUNK)), 2)

    plsc.subcore_barrier()

    def issue_gather(c, p):
        pltpu.async_copy(x_hbm.at[slist.at[pl.ds(c * CHUNK, CHUNK)]],
                         rows_v.at[p], gsem[p])

    def wait_gather(p):
        pltpu.make_async_copy(x_hbm.at[slist.at[pl.ds(0, CHUNK)]],
                              rows_v.at[p], gsem[p]).wait()

    def issue_scatter(c, p):
        for j in range(CHUNK // 16):
            dstb[p, pl.ds(j * 16, 16)] = dlist[pl.ds(c * CHUNK + j * 16, 16)]
        pltpu.async_copy(rows_v.at[p], agg_sh.at[dstb.at[p]],
                         ssem[p], add=True)

    def wait_scatter(p):
        pltpu.make_async_copy(rows_v.at[p], agg_sh.at[dstb.at[0]],
                              ssem[p]).wait()

    issue_gather(0, 0)

    def _body(t, carry):
        for j in range(2):
            c = 2 * t + j
            p = j

            @pl.when(c < nchunks)
            def _step():
                wait_gather(p)
                issue_scatter(c, p)

                @pl.when(c + 1 < nchunks)
                def _ahead():
                    @pl.when(c >= 1)
                    def _drain():
                        wait_scatter(1 - p)

                    issue_gather(c + 1, 1 - p)
        return carry

    lax.fori_loop(0, MAXC // 2, _body, 0)

    wait_scatter(0)
    wait_scatter(1)

    plsc.subcore_barrier()

    @pl.when(sid < 8)
    def _copy_out():
        obase = sid * 632
        pltpu.sync_copy(agg_sh.at[pl.ds(obase, 632)],
                        out_hbm.at[pl.ds(cid * SPAN + obase, 632)])


def _tc_body(p_ref, w_ref, b_ref, o_ref):
    y = lax.dot_general(p_ref[...], w_ref[...], (((1,), (1,)), ((), ())),
                        preferred_element_type=jnp.float32)
    o_ref[...] = jnp.maximum(y + b_ref[...], 0.0)


_BLK = 2000

_tc_apply = pl.pallas_call(
    _tc_body,
    grid=(N_NODES // _BLK,),
    in_specs=[
        pl.BlockSpec((_BLK, D), lambda i: (i, 0)),
        pl.BlockSpec((D, D), lambda i: (0, 0)),
        pl.BlockSpec((1, D), lambda i: (0, 0)),
    ],
    out_specs=pl.BlockSpec((_BLK, D), lambda i: (i, 0)),
    out_shape=jax.ShapeDtypeStruct((N_NODES, D), jnp.float32),
)


def kernel(x, edge_index, W, b):
    src = edge_index[0].astype(jnp.int32)
    dst = edge_index[1].astype(jnp.int32)
    pad = E_PAD - N_EDGES
    src = jnp.concatenate([src, jnp.zeros((pad,), jnp.int32)])
    dst = jnp.concatenate([dst, jnp.full((pad,), N_NODES, jnp.int32)])
    agg = _sc_aggregate(x, src, dst)
    return _tc_apply(agg, W, b.reshape(1, D))


# speedup vs baseline: 1.3330x; 1.3330x over previous
import functools

import jax
import jax.numpy as jnp
from jax import lax
from jax.experimental import pallas as pl
from jax.experimental.pallas import tpu as pltpu
from jax.experimental.pallas import tpu_sc as plsc

N_NODES = 10000
D = 128
N_EDGES = 320000

NC = 2
NS = 16
NW = NC * NS

CHUNK = 128
EPW = 10240
NCHUNK = EPW // CHUNK
E_PAD = NW * EPW
UNROLL = 4

AGG_ROWS = 10112
ZROWS = AGG_ROWS // NS
OROWS = ZROWS

_mesh = plsc.VectorSubcoreMesh(core_axis_name="c", subcore_axis_name="s")


@functools.partial(
    pl.kernel,
    out_type=jax.ShapeDtypeStruct((NC, AGG_ROWS, D), jnp.float32),
    mesh=_mesh,
    compiler_params=pltpu.CompilerParams(needs_layout_passes=False),
    scratch_types=[
        pltpu.VMEM((UNROLL, CHUNK), jnp.int32),
        pltpu.VMEM((UNROLL, CHUNK), jnp.int32),
        pltpu.VMEM((2, CHUNK, D), jnp.float32),
        pltpu.VMEM_SHARED((AGG_ROWS, D), jnp.float32),
        pltpu.SemaphoreType.DMA,
        pltpu.SemaphoreType.DMA,
        pltpu.SemaphoreType.DMA,
        pltpu.SemaphoreType.DMA,
        pltpu.SemaphoreType.DMA,
    ],
)
def _sc_aggregate(x_hbm, src_hbm, dst_hbm, out_hbm,
                  srcb, dstb, rows_v, agg_sh,
                  gsem0, gsem1, ssem0, ssem1, isem):
    cid = lax.axis_index("c")
    sid = lax.axis_index("s")
    wid = sid * NC + cid
    gsem = (gsem0, gsem1)
    ssem = (ssem0, ssem1)
    ebase = wid * EPW

    zero16 = jnp.zeros((16,), jnp.float32)

    def _zero_row(r, carry):
        for j in range(D // 16):
            rows_v[0, r, pl.ds(j * 16, 16)] = zero16
        return carry

    lax.fori_loop(0, CHUNK, _zero_row, 0)

    zbase = sid * ZROWS
    zfull, zrem = divmod(ZROWS, CHUNK)
    for k in range(zfull):
        pltpu.sync_copy(rows_v.at[0],
                        agg_sh.at[pl.ds(zbase + k * CHUNK, CHUNK)])
    if zrem:
        pltpu.sync_copy(rows_v.at[0, pl.ds(0, zrem)],
                        agg_sh.at[pl.ds(zbase + zfull * CHUNK, zrem)])

    plsc.subcore_barrier()

    def issue_idx(c, r):
        off = ebase + c * CHUNK
        pltpu.async_copy(src_hbm.at[pl.ds(off, CHUNK)], srcb.at[r], isem)
        pltpu.async_copy(dst_hbm.at[pl.ds(off, CHUNK)], dstb.at[r], isem)

    def wait_idx():
        pltpu.make_async_copy(src_hbm.at[pl.ds(0, CHUNK)],
                              srcb.at[0], isem).wait()
        pltpu.make_async_copy(dst_hbm.at[pl.ds(0, CHUNK)],
                              dstb.at[0], isem).wait()

    def issue_gather(c, r, p):
        pltpu.async_copy(x_hbm.at[srcb.at[r]], rows_v.at[p], gsem[p])

    def wait_gather(p):
        pltpu.make_async_copy(x_hbm.at[srcb.at[0]],
                              rows_v.at[p], gsem[p]).wait()

    def issue_scatter(r, p):
        pltpu.async_copy(rows_v.at[p], agg_sh.at[dstb.at[r]],
                         ssem[p], add=True)

    def wait_scatter(p):
        pltpu.make_async_copy(rows_v.at[p], agg_sh.at[dstb.at[0]],
                              ssem[p]).wait()

    pltpu.sync_copy(src_hbm.at[pl.ds(ebase, CHUNK)], srcb.at[0])
    pltpu.sync_copy(dst_hbm.at[pl.ds(ebase, CHUNK)], dstb.at[0])
    issue_gather(0, 0, 0)
    issue_idx(1, 1)

    def _body(t, carry):
        for j in range(UNROLL):
            c = t * UNROLL + j
            p = j % 2
            wait_gather(p)
            issue_scatter(j, p)
            wait_idx()

            @pl.when(c >= 1)
            def _drain():
                wait_scatter(1 - p)

            issue_gather(c + 1, (j + 1) % UNROLL, 1 - p)
            issue_idx(c + 2, (j + 2) % UNROLL)
        return carry

    lax.fori_loop(0, NCHUNK // UNROLL, _body, 0)

    wait_gather(NCHUNK % 2)
    wait_idx()
    wait_scatter((NCHUNK - 1) % 2)

    plsc.subcore_barrier()

    obase = sid * OROWS
    pltpu.sync_copy(agg_sh.at[pl.ds(obase, OROWS)],
                    out_hbm.at[cid, pl.ds(obase, OROWS)])


def _tc_body(p_ref, w_ref, b_ref, o_ref):
    acc = p_ref[0] + p_ref[1]
    y = lax.dot_general(acc, w_ref[...], (((1,), (1,)), ((), ())),
                        preferred_element_type=jnp.float32)
    o_ref[...] = jnp.maximum(y + b_ref[...], 0.0)


_BLK = 2000

_tc_apply = pl.pallas_call(
    _tc_body,
    grid=(N_NODES // _BLK,),
    in_specs=[
        pl.BlockSpec((NC, _BLK, D), lambda i: (0, i, 0)),
        pl.BlockSpec((D, D), lambda i: (0, 0)),
        pl.BlockSpec((1, D), lambda i: (0, 0)),
    ],
    out_specs=pl.BlockSpec((_BLK, D), lambda i: (i, 0)),
    out_shape=jax.ShapeDtypeStruct((N_NODES, D), jnp.float32),
)


def kernel(x, edge_index, W, b):
    src = edge_index[0].astype(jnp.int32)
    dst = edge_index[1].astype(jnp.int32)
    pad = E_PAD - N_EDGES
    over = 2 * CHUNK
    src = jnp.concatenate([src, jnp.zeros((pad + over,), jnp.int32)])
    dst = jnp.concatenate([dst, jnp.full((pad,), N_NODES, jnp.int32),
                           jnp.zeros((over,), jnp.int32)])
    partials = _sc_aggregate(x, src, dst)
    return _tc_apply(partials, W, b.reshape(1, D))
